# Initial kernel scaffold; baseline (speedup 1.0000x reference)
import jax
import jax.numpy as jnp
from jax.experimental import pallas as pl


def kernel(node_emb, Wl1, bl1, Wr1, Wl2, bl2, Wr2, Wm1, bm1, Wm2, bm2, edge_index, edge_index_for_loss):
    raise NotImplementedError("write your pallas kernel here")


# speedup vs baseline: 4.6019x; 4.6019x over previous
import functools

import jax
import jax.numpy as jnp
from jax import lax
from jax.experimental import pallas as pl
from jax.experimental.pallas import tpu as pltpu
from jax.experimental.pallas import tpu_sc as plsc

H = 128
NP = 10240
CHUNK = 128
NC = 2
NS = 16
NW = NC * NS
RPS = NP // NS
L = 16

_MESH = plsc.VectorSubcoreMesh(core_axis_name="c", subcore_axis_name="s")


@functools.cache
def _make_segsum(n_edges):
    tot_chunks = n_edges // CHUNK
    k_iters = -(-tot_chunks // NW)

    @functools.partial(
        pl.kernel,
        out_type=jax.ShapeDtypeStruct((NC, NP, H), jnp.float32),
        mesh=_MESH,
        scratch_types=[
            pltpu.VMEM((CHUNK,), jnp.int32),
            pltpu.VMEM((CHUNK,), jnp.int32),
            pltpu.VMEM((CHUNK, H), jnp.float32),
            pltpu.SemaphoreType.DMA,
            pltpu.VMEM_SHARED((NP, H), jnp.float32),
        ],
    )
    def segsum(x, src, dst, zf, agg, idx_s, idx_d, rows, sem, acc):
        cid = lax.axis_index("c")
        sid = lax.axis_index("s")
        wid = sid * NC + cid
        r0 = sid * RPS
        pltpu.sync_copy(zf, acc.at[pl.ds(r0, RPS)])
        plsc.subcore_barrier()

        @pl.loop(0, k_iters)
        def _(k):
            c = k * NW + wid

            @pl.when(c < tot_chunks)
            def _():
                base = c * CHUNK
                pltpu.sync_copy(src.at[pl.ds(base, CHUNK)], idx_s)
                pltpu.async_copy(x.at[idx_s], rows, sem).wait()
                pltpu.sync_copy(dst.at[pl.ds(base, CHUNK)], idx_d)
                pltpu.sync_copy(rows, acc.at[idx_d], add=True)

        plsc.subcore_barrier()
        pltpu.sync_copy(acc.at[pl.ds(r0, RPS)], agg.at[cid, pl.ds(r0, RPS)])

    return segsum


@functools.cache
def _make_degree(n_edges):
    tot_chunks = n_edges // CHUNK
    k_iters = -(-tot_chunks // NW)

    @functools.partial(
        pl.kernel,
        out_type=jax.ShapeDtypeStruct((NC, NP), jnp.float32),
        mesh=_MESH,
        compiler_params=pltpu.CompilerParams(needs_layout_passes=False),
        scratch_types=[
            pltpu.VMEM((CHUNK,), jnp.int32),
            pltpu.VMEM((NP,), jnp.float32),
            pltpu.VMEM((NS, RPS), jnp.float32),
            pltpu.VMEM_SHARED((NS, NP), jnp.float32),
        ],
    )
    def degree(dst, z1, deg, idx_d, hist, buf, hist_sh):
        cid = lax.axis_index("c")
        sid = lax.axis_index("s")
        wid = sid * NC + cid
        r0 = sid * RPS
        pltpu.sync_copy(z1, hist)
        ones = jnp.ones((L,), jnp.float32)

        @pl.loop(0, k_iters)
        def _(k):
            c = k * NW + wid

            @pl.when(c < tot_chunks)
            def _():
                base = c * CHUNK
                pltpu.sync_copy(dst.at[pl.ds(base, CHUNK)], idx_d)
                for j in range(CHUNK // L):
                    iv = idx_d[pl.ds(j * L, L)]
                    plsc.addupdate_scatter(hist, [iv], ones)

        pltpu.sync_copy(hist, hist_sh.at[sid])
        plsc.subcore_barrier()
        pltpu.sync_copy(hist_sh.at[:, pl.ds(r0, RPS)], buf)
        for c in range(RPS // L):
            sl = pl.ds(c * L, L)
            tot = buf[0, sl]
            for r in range(1, NS):
                tot = tot + buf[r, sl]
            hist[sl] = tot
        pltpu.sync_copy(hist.at[pl.ds(0, RPS)], deg.at[cid, pl.ds(r0, RPS)])

    return degree


@functools.cache
def _make_edge_gather(n_edges):
    tot_chunks = n_edges // CHUNK
    k_iters = -(-tot_chunks // NW)

    @functools.partial(
        pl.kernel,
        out_type=(
            jax.ShapeDtypeStruct((n_edges, H), jnp.float32),
            jax.ShapeDtypeStruct((n_edges, H), jnp.float32),
        ),
        mesh=_MESH,
        scratch_types=[
            pltpu.VMEM((CHUNK,), jnp.int32),
            pltpu.VMEM((CHUNK,), jnp.int32),
            pltpu.VMEM((CHUNK, H), jnp.float32),
            pltpu.VMEM((CHUNK, H), jnp.float32),
            pltpu.SemaphoreType.DMA,
            pltpu.SemaphoreType.DMA,
        ],
    )
    def edge_gather(a_hbm, b_hbm, srcl, dstl, oa, ob, idx_s, idx_d, ga, gb, sa, sb):
        cid = lax.axis_index("c")
        sid = lax.axis_index("s")
        wid = sid * NC + cid

        @pl.loop(0, k_iters)
        def _(k):
            c = k * NW + wid

            @pl.when(c < tot_chunks)
            def _():
                base = c * CHUNK
                pltpu.sync_copy(srcl.at[pl.ds(base, CHUNK)], idx_s)
                pltpu.sync_copy(dstl.at[pl.ds(base, CHUNK)], idx_d)
                cpa = pltpu.async_copy(a_hbm.at[idx_s], ga, sa)
                cpb = pltpu.async_copy(b_hbm.at[idx_d], gb, sb)
                cpa.wait()
                cpb.wait()
                pltpu.sync_copy(ga, oa.at[pl.ds(base, CHUNK)])
                pltpu.sync_copy(gb, ob.at[pl.ds(base, CHUNK)])

    return edge_gather


RB = 640
RB3 = 4000


def _combine1_body(agg_ref, deg_ref, x_ref, wl_ref, bl_ref, wr_ref, out_ref):
    agg = agg_ref[...]
    agg = agg[0] + agg[1]
    d = deg_ref[...]
    deg = jnp.maximum(d[0] + d[1], 1.0)
    mean = agg / deg
    out_ref[...] = jnp.maximum(
        jnp.dot(mean, wl_ref[...], preferred_element_type=jnp.float32)
        + bl_ref[...]
        + jnp.dot(x_ref[...], wr_ref[...], preferred_element_type=jnp.float32),
        0.0,
    )


def _combine2_body(agg_ref, deg_ref, x_ref, wl_ref, bl_ref, wr_ref, wma_ref,
                   wmb_ref, bm_ref, a_ref, b_ref):
    agg = agg_ref[...]
    agg = agg[0] + agg[1]
    d = deg_ref[...]
    deg = jnp.maximum(d[0] + d[1], 1.0)
    mean = agg / deg
    h = jnp.maximum(
        jnp.dot(mean, wl_ref[...], preferred_element_type=jnp.float32)
        + bl_ref[...]
        + jnp.dot(x_ref[...], wr_ref[...], preferred_element_type=jnp.float32),
        0.0,
    )
    a_ref[...] = (jnp.dot(h, wma_ref[...], preferred_element_type=jnp.float32)
                  + bm_ref[...])
    b_ref[...] = jnp.dot(h, wmb_ref[...], preferred_element_type=jnp.float32)


def _head_body(a_ref, b_ref, w_ref, b2_ref, out_ref):
    h = jnp.maximum(a_ref[...] + b_ref[...], 0.0)
    out_ref[...] = (jnp.dot(h, w_ref[...], preferred_element_type=jnp.float32)
                    + b2_ref[...])


def _combine1(agg, deg, x_pad, wl, bl, wr):
    return pl.pallas_call(
        _combine1_body,
        grid=(NP // RB,),
        in_specs=[
            pl.BlockSpec((NC, RB, H), lambda i: (0, i, 0)),
            pl.BlockSpec((NC, RB, 1), lambda i: (0, i, 0)),
            pl.BlockSpec((RB, H), lambda i: (i, 0)),
            pl.BlockSpec((H, H), lambda i: (0, 0)),
            pl.BlockSpec((1, H), lambda i: (0, 0)),
            pl.BlockSpec((H, H), lambda i: (0, 0)),
        ],
        out_specs=pl.BlockSpec((RB, H), lambda i: (i, 0)),
        out_shape=jax.ShapeDtypeStruct((NP, H), jnp.float32),
    )(agg, deg, x_pad, wl, bl.reshape(1, H), wr)


def _combine2(agg, deg, x1, wl, bl, wr, wma, wmb, bm):
    return pl.pallas_call(
        _combine2_body,
        grid=(NP // RB,),
        in_specs=[
            pl.BlockSpec((NC, RB, H), lambda i: (0, i, 0)),
            pl.BlockSpec((NC, RB, 1), lambda i: (0, i, 0)),
            pl.BlockSpec((RB, H), lambda i: (i, 0)),
            pl.BlockSpec((H, H), lambda i: (0, 0)),
            pl.BlockSpec((1, H), lambda i: (0, 0)),
            pl.BlockSpec((H, H), lambda i: (0, 0)),
            pl.BlockSpec((H, H), lambda i: (0, 0)),
            pl.BlockSpec((H, H), lambda i: (0, 0)),
            pl.BlockSpec((1, H), lambda i: (0, 0)),
        ],
        out_specs=[
            pl.BlockSpec((RB, H), lambda i: (i, 0)),
            pl.BlockSpec((RB, H), lambda i: (i, 0)),
        ],
        out_shape=[
            jax.ShapeDtypeStruct((NP, H), jnp.float32),
            jax.ShapeDtypeStruct((NP, H), jnp.float32),
        ],
    )(agg, deg, x1, wl, bl.reshape(1, H), wr, wma, wmb, bm.reshape(1, H))


def _head(ga, gb, wm2, bm2):
    n_edges = ga.shape[0]
    ncls = wm2.shape[1]
    return pl.pallas_call(
        _head_body,
        grid=(n_edges // RB3,),
        in_specs=[
            pl.BlockSpec((RB3, H), lambda i: (i, 0)),
            pl.BlockSpec((RB3, H), lambda i: (i, 0)),
            pl.BlockSpec((H, ncls), lambda i: (0, 0)),
            pl.BlockSpec((1, ncls), lambda i: (0, 0)),
        ],
        out_specs=pl.BlockSpec((RB3, ncls), lambda i: (i, 0)),
        out_shape=jax.ShapeDtypeStruct((n_edges, ncls), jnp.float32),
    )(ga, gb, wm2, bm2.reshape(1, ncls))


def kernel(node_emb, Wl1, bl1, Wr1, Wl2, bl2, Wr2, Wm1, bm1, Wm2, bm2,
           edge_index, edge_index_for_loss):
    n = node_emb.shape[0]
    x_pad = jnp.pad(node_emb, ((0, NP - n), (0, 0)))
    zf = jnp.zeros((RPS, H), jnp.float32)
    z1 = jnp.zeros((NP,), jnp.float32)
    src = edge_index[0]
    dst = edge_index[1]
    srcl = edge_index_for_loss[0]
    dstl = edge_index_for_loss[1]

    n_edges = edge_index.shape[1]
    segsum = _make_segsum(n_edges)
    deg = _make_degree(n_edges)(dst, z1).reshape(NC, NP, 1)
    agg1 = segsum(x_pad, src, dst, zf)
    x1 = _combine1(agg1, deg, x_pad, Wl1, bl1, Wr1)
    agg2 = segsum(x1, src, dst, zf)
    a_tab, b_tab = _combine2(agg2, deg, x1, Wl2, bl2, Wr2, Wm1[:H], Wm1[H:], bm1)
    ga, gb = _make_edge_gather(edge_index_for_loss.shape[1])(
        a_tab, b_tab, srcl, dstl)
    return _head(ga, gb, Wm2, bm2)
